# Initial kernel scaffold; baseline (speedup 1.0000x reference)
import functools

import jax
import jax.numpy as jnp
from jax import lax
from jax.experimental import pallas as pl
from jax.experimental.pallas import tpu as pltpu
from jax.experimental.pallas import tpu_sc as plsc

NC = 2
NS = 16
LANES = 16
CHUNK = 128


def _sc_partials(x, src, dst, w):
    n, d = x.shape
    e = src.shape[0]
    nw = NC * NS
    n_chunks = e // CHUNK
    assert n_chunks * CHUNK == e
    full_rounds = n_chunks // nw
    rem = n_chunks % nw
    rows_per_tile = n // NS
    wchunk = 125
    assert rows_per_tile % wchunk == 0
    nvec = d // LANES

    mesh = plsc.VectorSubcoreMesh(core_axis_name="c", subcore_axis_name="s")

    @functools.partial(
        pl.kernel,
        out_type=jax.ShapeDtypeStruct((NC, n, d), jnp.float32),
        mesh=mesh,
        scratch_types=[
            pltpu.VMEM((1, CHUNK, d), jnp.float32),
            pltpu.VMEM((1, CHUNK), jnp.int32),
            pltpu.VMEM((1, CHUNK), jnp.int32),
            pltpu.VMEM((1, CHUNK), jnp.float32),
            pltpu.VMEM_SHARED((n, d), jnp.float32),
            pltpu.SemaphoreType.DMA,
        ],
    )
    def sc_kernel(x_hbm, src_hbm, dst_hbm, w_hbm, out_hbm, rows, srcb, dstb,
                  wb, acc, sem):
        cid = lax.axis_index("c")
        sid = lax.axis_index("s")
        wid = sid * NC + cid

        zeros = jnp.zeros((LANES,), jnp.float32)

        def zero_row(i, carry):
            for f in range(nvec):
                rows[0, i, pl.ds(LANES * f, LANES)] = zeros
            return carry

        lax.fori_loop(0, CHUNK, zero_row, 0)

        base = sid * rows_per_tile
        for j in range(rows_per_tile // wchunk):
            pltpu.sync_copy(rows.at[0, pl.ds(0, wchunk)],
                            acc.at[pl.ds(base + j * wchunk, wchunk)])
        plsc.subcore_barrier()

        def do_chunk(c):
            off = c * CHUNK
            pltpu.sync_copy(src_hbm.at[pl.ds(off, CHUNK)], srcb.at[0])
            pltpu.sync_copy(dst_hbm.at[pl.ds(off, CHUNK)], dstb.at[0])
            pltpu.sync_copy(w_hbm.at[pl.ds(off, CHUNK)], wb.at[0])
            pltpu.async_copy(x_hbm.at[srcb.at[0]], rows.at[0], sem).wait()

            def scale_row(i, carry):
                idx16 = jnp.full((LANES,), i, jnp.int32)
                wsplat = plsc.load_gather(wb.at[0], [idx16])
                for f in range(nvec):
                    sl = (0, i, pl.ds(LANES * f, LANES))
                    rows[sl] = rows[sl] * wsplat
                return carry

            lax.fori_loop(0, CHUNK, scale_row, 0)
            pltpu.sync_copy(rows.at[0], acc.at[dstb.at[0]], add=True)

        def round_body(j, carry):
            do_chunk(j * nw + wid)
            return carry

        lax.fori_loop(0, full_rounds, round_body, 0)
        if rem:
            @pl.when(wid < rem)
            def _():
                do_chunk(full_rounds * nw + wid)

        plsc.subcore_barrier()

        for j in range(rows_per_tile // wchunk):
            r0 = base + j * wchunk
            pltpu.sync_copy(acc.at[pl.ds(r0, wchunk)],
                            rows.at[0, pl.ds(0, wchunk)])
            pltpu.sync_copy(rows.at[0, pl.ds(0, wchunk)],
                            out_hbm.at[cid, pl.ds(r0, wchunk)])

    return sc_kernel(x, src, dst, w)


def _combine_body(p_ref, o_ref):
    o_ref[...] = p_ref[0] + p_ref[1]


def _combine(partials):
    nc, n, d = partials.shape
    blk = 1000
    return pl.pallas_call(
        _combine_body,
        grid=(n // blk,),
        in_specs=[pl.BlockSpec((nc, blk, d), lambda i: (0, i, 0))],
        out_specs=pl.BlockSpec((blk, d), lambda i: (i, 0)),
        out_shape=jax.ShapeDtypeStruct((n, d), jnp.float32),
    )(partials)


@jax.jit
def kernel(x, edge_index, edge_weights):
    src = edge_index[0]
    dst = edge_index[1]
    partials = _sc_partials(x, src, dst, edge_weights)
    return _combine(partials)


# speedup vs baseline: 4.8114x; 4.8114x over previous
import functools

import jax
import jax.numpy as jnp
from jax import lax
from jax.experimental import pallas as pl
from jax.experimental.pallas import tpu as pltpu
from jax.experimental.pallas import tpu_sc as plsc

NC = 2
NS = 16
LANES = 16
CHUNK = 128


def _sc_partials(x, src, dst, w):
    n, d = x.shape
    e = src.shape[0]
    nw = NC * NS
    n_chunks = e // CHUNK
    assert n_chunks * CHUNK == e
    full_rounds = n_chunks // nw
    rem = n_chunks % nw
    wchunk = 80
    assert n % wchunk == 0 and wchunk % 8 == 0 and wchunk <= CHUNK
    n_spans = n // wchunk
    span_rounds = -(-n_spans // NS)
    nvec = d // LANES

    mesh = plsc.VectorSubcoreMesh(core_axis_name="c", subcore_axis_name="s")

    @functools.partial(
        pl.kernel,
        out_type=jax.ShapeDtypeStruct((NC, n, d), jnp.float32),
        mesh=mesh,
        scratch_types=[
            pltpu.VMEM((1, CHUNK, d), jnp.float32),
            pltpu.VMEM((1, CHUNK), jnp.int32),
            pltpu.VMEM((1, CHUNK), jnp.int32),
            pltpu.VMEM((1, CHUNK + LANES), jnp.float32),
            pltpu.VMEM_SHARED((n, d), jnp.float32),
            pltpu.SemaphoreType.DMA,
        ],
    )
    def sc_kernel(x_hbm, src_hbm, dst_hbm, w_hbm, out_hbm, rows, srcb, dstb,
                  wb, acc, sem):
        cid = lax.axis_index("c")
        sid = lax.axis_index("s")
        wid = sid * NC + cid

        zeros = jnp.zeros((LANES,), jnp.float32)

        def zero_row(i, carry):
            for f in range(nvec):
                rows[0, i, pl.ds(LANES * f, LANES)] = zeros
            return carry

        lax.fori_loop(0, CHUNK, zero_row, 0)

        for j in range(span_rounds):
            c = j * NS + sid

            @pl.when(c < n_spans)
            def _():
                pltpu.sync_copy(rows.at[0, pl.ds(0, wchunk)],
                                acc.at[pl.ds(c * wchunk, wchunk)])
        plsc.subcore_barrier()

        def do_chunk(c):
            off = c * CHUNK
            pltpu.sync_copy(src_hbm.at[pl.ds(off, CHUNK)], srcb.at[0])
            pltpu.sync_copy(dst_hbm.at[pl.ds(off, CHUNK)], dstb.at[0])
            pltpu.sync_copy(w_hbm.at[pl.ds(off, CHUNK)],
                            wb.at[0, pl.ds(0, CHUNK)])
            pltpu.async_copy(x_hbm.at[srcb.at[0]], rows.at[0], sem).wait()

            def scale_row(i, carry):
                wrow = wb[0, pl.ds(i, LANES)]
                wsplat = jnp.full((LANES,), wrow[0])
                for f in range(nvec):
                    sl = (0, i, pl.ds(LANES * f, LANES))
                    rows[sl] = rows[sl] * wsplat
                return carry

            lax.fori_loop(0, CHUNK, scale_row, 0)
            pltpu.sync_copy(rows.at[0], acc.at[dstb.at[0]], add=True)

        def round_body(j, carry):
            do_chunk(j * nw + wid)
            return carry

        lax.fori_loop(0, full_rounds, round_body, 0)
        if rem:
            @pl.when(wid < rem)
            def _():
                do_chunk(full_rounds * nw + wid)

        plsc.subcore_barrier()

        for j in range(span_rounds):
            c = j * NS + sid

            @pl.when(c < n_spans)
            def _():
                r0 = c * wchunk
                pltpu.sync_copy(acc.at[pl.ds(r0, wchunk)],
                                rows.at[0, pl.ds(0, wchunk)])
                pltpu.sync_copy(rows.at[0, pl.ds(0, wchunk)],
                                out_hbm.at[cid, pl.ds(r0, wchunk)])

    return sc_kernel(x, src, dst, w)


def _combine_body(p_ref, o_ref):
    o_ref[...] = p_ref[0] + p_ref[1]


def _combine(partials):
    nc, n, d = partials.shape
    blk = 1000
    return pl.pallas_call(
        _combine_body,
        grid=(n // blk,),
        in_specs=[pl.BlockSpec((nc, blk, d), lambda i: (0, i, 0))],
        out_specs=pl.BlockSpec((blk, d), lambda i: (i, 0)),
        out_shape=jax.ShapeDtypeStruct((n, d), jnp.float32),
    )(partials)


@jax.jit
def kernel(x, edge_index, edge_weights):
    src = edge_index[0]
    dst = edge_index[1]
    partials = _sc_partials(x, src, dst, edge_weights)
    return _combine(partials)


# speedup vs baseline: 6.1896x; 1.2865x over previous
import functools

import jax
import jax.numpy as jnp
from jax import lax
from jax.experimental import pallas as pl
from jax.experimental.pallas import tpu as pltpu
from jax.experimental.pallas import tpu_sc as plsc

NC = 2
NS = 16
LANES = 16
CHUNK = 40
NBUF = 5
WGRP = 8


def _sc_partials(x, src, dst, w):
    n, d = x.shape
    e = src.shape[0]
    nw = NC * NS
    e_per_w = e // nw
    assert e_per_w * nw == e and e_per_w % CHUNK == 0 and CHUNK % 8 == 0
    n_chunks = e_per_w // CHUNK
    assert n_chunks % NBUF == 0
    n_rounds = n_chunks // NBUF
    n_spans = n // CHUNK
    assert n % CHUNK == 0
    span_rounds = -(-n_spans // NS)
    nvec = d // LANES
    egrp = CHUNK // WGRP
    wpad = CHUNK + LANES

    mesh = plsc.VectorSubcoreMesh(core_axis_name="c", subcore_axis_name="s")

    @functools.partial(
        pl.kernel,
        out_type=jax.ShapeDtypeStruct((NC, n, d), jnp.float32),
        mesh=mesh,
        scratch_types=[
            pltpu.VMEM((NBUF, CHUNK, d), jnp.float32),
            pltpu.VMEM((NBUF, CHUNK), jnp.int32),
            pltpu.VMEM((NBUF, CHUNK), jnp.int32),
            pltpu.VMEM((NBUF, wpad), jnp.float32),
            pltpu.VMEM_SHARED((n, d), jnp.float32),
            pltpu.SemaphoreType.DMA((NBUF,)),
            pltpu.SemaphoreType.DMA((NBUF,)),
            pltpu.SemaphoreType.DMA((NBUF,)),
            pltpu.SemaphoreType.DMA((NBUF,)),
        ],
    )
    def sc_kernel(x_hbm, src_hbm, dst_hbm, w_hbm, out_hbm, rows, srcb, dstb,
                  wb, acc, sem_g, sem_s, sem_sw, sem_d):
        cid = lax.axis_index("c")
        sid = lax.axis_index("s")
        wid = sid * NC + cid
        ebase = wid * e_per_w

        zeros = jnp.zeros((LANES,), jnp.float32)

        def zero_row(i, carry):
            for f in range(nvec):
                rows[0, i, pl.ds(LANES * f, LANES)] = zeros
            return carry

        lax.fori_loop(0, CHUNK, zero_row, 0)

        for j in range(span_rounds):
            c = j * NS + sid

            @pl.when(c < n_spans)
            def _():
                pltpu.sync_copy(rows.at[0], acc.at[pl.ds(c * CHUNK, CHUNK)])
        plsc.subcore_barrier()

        def src_copy(j, b):
            return pltpu.make_async_copy(
                src_hbm.at[pl.ds(ebase + j * CHUNK, CHUNK)], srcb.at[b],
                sem_sw.at[b])

        def w_copy(j, b):
            return pltpu.make_async_copy(
                w_hbm.at[pl.ds(ebase + j * CHUNK, CHUNK)],
                wb.at[b, pl.ds(0, CHUNK)], sem_sw.at[b])

        def dst_copy(j, b):
            return pltpu.make_async_copy(
                dst_hbm.at[pl.ds(ebase + j * CHUNK, CHUNK)], dstb.at[b],
                sem_d.at[b])

        def gather_copy(j, b):
            return pltpu.make_async_copy(
                x_hbm.at[srcb.at[b]], rows.at[b], sem_g.at[b])

        def scale_chunk(b):
            rows_b = rows.at[b]

            def grp(g, carry):
                w16 = wb[b, pl.ds(WGRP * g, LANES)]
                for ee in range(WGRP):
                    wsplat = jnp.full((LANES,), w16[ee])
                    i = WGRP * g + ee
                    for f in range(nvec):
                        sl = (i, pl.ds(LANES * f, LANES))
                        rows_b[sl] = rows_b[sl] * wsplat
                return carry

            lax.fori_loop(0, egrp, grp, 0)

        for c in range(2):
            src_copy(c, c).start()
            w_copy(c, c).start()
            dst_copy(c, c).start()
        for c in range(2, 4):
            src_copy(c, c).start()
            w_copy(c, c).start()
        for c in range(2):
            src_copy(c, c).wait()
            gather_copy(c, c).start()

        def round_body(q, carry):
            for k in range(NBUF):
                j = q * NBUF + k
                b2 = (k + 2) % NBUF
                b4 = (k + 4) % NBUF

                @pl.when(j >= 3)
                def _():
                    pltpu.make_async_copy(
                        rows.at[b2], acc.at[dstb.at[b2]], sem_s.at[b2]).wait()

                @pl.when(j + 2 < n_chunks)
                def _():
                    dst_copy(j + 2, b2).start()

                @pl.when(j + 4 < n_chunks)
                def _():
                    src_copy(j + 4, b4).start()
                    w_copy(j + 4, b4).start()

                @pl.when(j + 2 < n_chunks)
                def _():
                    src_copy(j + 2, b2).wait()
                    gather_copy(j + 2, b2).start()

                gather_copy(j, k).wait()
                w_copy(j, k).wait()
                scale_chunk(k)
                dst_copy(j, k).wait()
                pltpu.async_copy(rows.at[k], acc.at[dstb.at[k]], sem_s.at[k],
                                 add=True)
            return carry

        lax.fori_loop(0, n_rounds, round_body, 0)
        for c in range(n_chunks - 3, n_chunks):
            k = c % NBUF
            pltpu.make_async_copy(
                rows.at[k], acc.at[dstb.at[k]], sem_s.at[k]).wait()

        plsc.subcore_barrier()

        for j in range(span_rounds):
            c = j * NS + sid

            @pl.when(c < n_spans)
            def _():
                r0 = c * CHUNK
                pltpu.sync_copy(acc.at[pl.ds(r0, CHUNK)], rows.at[0])
                pltpu.sync_copy(rows.at[0], out_hbm.at[cid, pl.ds(r0, CHUNK)])

    return sc_kernel(x, src, dst, w)


def _combine_body(p_ref, o_ref):
    o_ref[...] = p_ref[0] + p_ref[1]


def _combine(partials):
    nc, n, d = partials.shape
    blk = 1000
    return pl.pallas_call(
        _combine_body,
        grid=(n // blk,),
        in_specs=[pl.BlockSpec((nc, blk, d), lambda i: (0, i, 0))],
        out_specs=pl.BlockSpec((blk, d), lambda i: (i, 0)),
        out_shape=jax.ShapeDtypeStruct((n, d), jnp.float32),
    )(partials)


@jax.jit
def kernel(x, edge_index, edge_weights):
    src = edge_index[0]
    dst = edge_index[1]
    partials = _sc_partials(x, src, dst, edge_weights)
    return _combine(partials)


# speedup vs baseline: 10.6131x; 1.7146x over previous
import functools

import jax
import jax.numpy as jnp
from jax import lax
from jax.experimental import pallas as pl
from jax.experimental.pallas import tpu as pltpu
from jax.experimental.pallas import tpu_sc as plsc

NC = 2
NS = 16
LANES = 16
CHUNK = 40
NBUF = 5
WGRP = 8


def _sc_partials(x, src, dst, w):
    n, d = x.shape
    e = src.shape[0]
    nw = NC * NS
    e_per_w = e // nw
    assert e_per_w * nw == e and e_per_w % CHUNK == 0 and CHUNK % 8 == 0
    n_chunks = e_per_w // CHUNK
    assert n_chunks % NBUF == 0
    n_rounds = n_chunks // NBUF
    n_spans = n // CHUNK
    assert n % CHUNK == 0
    span_rounds = -(-n_spans // NS)
    nvec = d // LANES
    egrp = CHUNK // WGRP
    wpad = CHUNK + LANES

    mesh = plsc.VectorSubcoreMesh(core_axis_name="c", subcore_axis_name="s")

    @functools.partial(
        pl.kernel,
        out_type=jax.ShapeDtypeStruct((NC, n, d), jnp.float32),
        mesh=mesh,
        scratch_types=[
            pltpu.VMEM((NBUF, CHUNK, d), jnp.float32),
            pltpu.VMEM((NBUF, CHUNK), jnp.int32),
            pltpu.VMEM((NBUF, CHUNK), jnp.int32),
            pltpu.VMEM((NBUF, wpad), jnp.float32),
            pltpu.VMEM_SHARED((n, d), jnp.float32),
            pltpu.SemaphoreType.DMA((NBUF,)),
            pltpu.SemaphoreType.DMA((NBUF,)),
            pltpu.SemaphoreType.DMA((NBUF,)),
            pltpu.SemaphoreType.DMA((NBUF,)),
        ],
    )
    def sc_kernel(x_hbm, src_hbm, dst_hbm, w_hbm, out_hbm, rows, srcb, dstb,
                  wb, acc, sem_g, sem_s, sem_sw, sem_d):
        cid = lax.axis_index("c")
        sid = lax.axis_index("s")
        wid = sid * NC + cid
        ebase = wid * e_per_w

        def src_copy(j, b):
            return pltpu.make_async_copy(
                src_hbm.at[pl.ds(ebase + j * CHUNK, CHUNK)], srcb.at[b],
                sem_sw.at[b])

        def w_copy(j, b):
            return pltpu.make_async_copy(
                w_hbm.at[pl.ds(ebase + j * CHUNK, CHUNK)],
                wb.at[b, pl.ds(0, CHUNK)], sem_sw.at[b])

        def dst_copy(j, b):
            return pltpu.make_async_copy(
                dst_hbm.at[pl.ds(ebase + j * CHUNK, CHUNK)], dstb.at[b],
                sem_d.at[b])

        def gather_copy(j, b):
            return pltpu.make_async_copy(
                x_hbm.at[srcb.at[b]], rows.at[b], sem_g.at[b])

        def scale_chunk(b):
            rows_b = rows.at[b]
            for g in range(egrp):
                w16 = wb[b, pl.ds(WGRP * g, LANES)]
                for ee in range(WGRP):
                    wsplat = jnp.full((LANES,), w16[ee])
                    i = WGRP * g + ee
                    for f in range(nvec):
                        sl = (i, pl.ds(LANES * f, LANES))
                        rows_b[sl] = rows_b[sl] * wsplat

        for c in range(2):
            src_copy(c, c).start()
            w_copy(c, c).start()
            dst_copy(c, c).start()
        for c in range(2, 4):
            src_copy(c, c).start()
            w_copy(c, c).start()

        zeros = jnp.zeros((LANES,), jnp.float32)

        def zero_row(i, carry):
            for f in range(nvec):
                rows[NBUF - 1, i, pl.ds(LANES * f, LANES)] = zeros
            return carry

        lax.fori_loop(0, CHUNK, zero_row, 0)

        for j in range(span_rounds):
            c = j * NS + sid

            @pl.when(c < n_spans)
            def _():
                pltpu.async_copy(rows.at[NBUF - 1],
                                 acc.at[pl.ds(c * CHUNK, CHUNK)],
                                 sem_s.at[0])
        for j in range(span_rounds):
            c = j * NS + sid

            @pl.when(c < n_spans)
            def _():
                pltpu.make_async_copy(
                    rows.at[NBUF - 1],
                    acc.at[pl.ds(c * CHUNK, CHUNK)], sem_s.at[0]).wait()
        plsc.subcore_barrier()

        for c in range(2):
            src_copy(c, c).wait()
            gather_copy(c, c).start()

        def round_body(q, carry):
            for k in range(NBUF):
                j = q * NBUF + k
                b2 = (k + 2) % NBUF
                b4 = (k + 4) % NBUF

                @pl.when(j >= 3)
                def _():
                    pltpu.make_async_copy(
                        rows.at[b2], acc.at[dstb.at[b2]], sem_s.at[b2]).wait()

                @pl.when(j + 2 < n_chunks)
                def _():
                    dst_copy(j + 2, b2).start()

                @pl.when(j + 4 < n_chunks)
                def _():
                    src_copy(j + 4, b4).start()
                    w_copy(j + 4, b4).start()

                @pl.when(j + 2 < n_chunks)
                def _():
                    src_copy(j + 2, b2).wait()
                    gather_copy(j + 2, b2).start()

                gather_copy(j, k).wait()
                w_copy(j, k).wait()
                scale_chunk(k)
                dst_copy(j, k).wait()
                pltpu.async_copy(rows.at[k], acc.at[dstb.at[k]], sem_s.at[k],
                                 add=True)
            return carry

        lax.fori_loop(0, n_rounds, round_body, 0)
        for c in range(n_chunks - 3, n_chunks):
            k = c % NBUF
            pltpu.make_async_copy(
                rows.at[k], acc.at[dstb.at[k]], sem_s.at[k]).wait()

        plsc.subcore_barrier()

        def stage_a(c, b):
            return pltpu.make_async_copy(
                acc.at[pl.ds(c * CHUNK, CHUNK)], rows.at[b], sem_g.at[b])

        def stage_b(c, b):
            return pltpu.make_async_copy(
                rows.at[b], out_hbm.at[cid, pl.ds(c * CHUNK, CHUNK)],
                sem_s.at[b])

        for j in range(span_rounds):
            c = j * NS + sid
            b = j % 2

            @pl.when(c < n_spans)
            def _():
                if j >= 2:
                    cprev = (j - 2) * NS + sid
                    stage_b(cprev, b).wait()
                stage_a(c, b).start()
                stage_a(c, b).wait()
                stage_b(c, b).start()
        for j in range(span_rounds - 2, span_rounds):
            c = j * NS + sid
            b = j % 2

            @pl.when(c < n_spans)
            def _():
                stage_b(c, b).wait()

    return sc_kernel(x, src, dst, w)


def _combine_body(p_ref, o_ref):
    o_ref[...] = p_ref[0] + p_ref[1]


def _combine(partials):
    nc, n, d = partials.shape
    blk = 1000
    return pl.pallas_call(
        _combine_body,
        grid=(n // blk,),
        in_specs=[pl.BlockSpec((nc, blk, d), lambda i: (0, i, 0))],
        out_specs=pl.BlockSpec((blk, d), lambda i: (i, 0)),
        out_shape=jax.ShapeDtypeStruct((n, d), jnp.float32),
    )(partials)


@jax.jit
def kernel(x, edge_index, edge_weights):
    src = edge_index[0]
    dst = edge_index[1]
    partials = _sc_partials(x, src, dst, edge_weights)
    return _combine(partials)
